# baseline (speedup 1.0000x reference)
import functools

import jax
import jax.numpy as jnp
from jax import lax
from jax.experimental import pallas as pl
from jax.experimental.pallas import tpu as pltpu
from jax.experimental.pallas import tpu_sc as plsc

N = 10000
E = 160000
DX = 128
M = 500
MP = 512
H1 = 128
F = 144
NB = 10
BN = N // NB
NTILES = 16
EPT = E // NTILES
K = 40
NCH = EPT // K
N_PAD = 10240
ROWS_PT = N_PAD // NTILES


def _dot_t(a, b):
    return lax.dot_general(a, b, (((1,), (1,)), ((), ())),
                           preferred_element_type=jnp.float32)


def _elu(x):
    return jnp.where(x > 0, x, jnp.exp(x) - 1.0)


def _l2norm(x):
    n = jnp.sqrt(jnp.sum(x * x, axis=-1, keepdims=True))
    return x / jnp.maximum(n, 1e-12)


def _tc_tables1(x_ref, wi_ref, avi_ref, wo_ref, avo_ref,
                feat_ref, scal_ref, fold_ref):
    xb = x_ref[...]
    wi = wi_ref[...]
    wo = wo_ref[...]
    avi = avi_ref[...].reshape(H1)
    avo = avo_ref[...].reshape(H1)
    xa_i = _dot_t(xb, wi[:, :DX])
    xb_i = _dot_t(xb, wi[:, DX:2 * DX])
    xa_o = _dot_t(xb, wo[:, :DX])
    xb_o = _dot_t(xb, wo[:, DX:2 * DX])
    sa_i = (xa_i * avi).reshape(-1, 2, 64).sum(-1)
    sb_i = (xb_i * avi).reshape(-1, 2, 64).sum(-1)
    sa_o = (xa_o * avo).reshape(-1, 2, 64).sum(-1)
    sb_o = (xb_o * avo).reshape(-1, 2, 64).sum(-1)
    zf = jnp.zeros((BN, F - H1 - 2), jnp.float32)
    zs = jnp.zeros((BN, 14), jnp.float32)
    feat_ref[0] = jnp.concatenate([xa_i, sa_i, zf], axis=1)
    feat_ref[1] = jnp.concatenate([xb_o, sb_o, zf], axis=1)
    scal_ref[0] = jnp.concatenate([sb_i, zs], axis=1)
    scal_ref[1] = jnp.concatenate([sa_o, zs], axis=1)
    fold_ref[0] = xb_i
    fold_ref[1] = xa_o


def _tc_typetables(g_ref, w1i_ref, av1i_ref, w1o_ref, av1o_ref, wrel_ref,
                   w2i_ref, av2i_ref, w2o_ref, av2o_ref, t1_ref, t2_ref):
    gb = g_ref[...]
    av1i = av1i_ref[...].reshape(H1)
    av1o = av1o_ref[...].reshape(H1)
    av2i = av2i_ref[...].reshape(H1)
    av2o = av2o_ref[...].reshape(H1)
    gc_i = _dot_t(gb, w1i_ref[...][:, 2 * DX:])
    gc_o = _dot_t(gb, w1o_ref[...][:, 2 * DX:])
    sr_i = (gc_i * av1i).reshape(-1, 2, 64).sum(-1)
    sr_o = (gc_o * av1o).reshape(-1, 2, 64).sum(-1)
    zf = jnp.zeros((MP, F - H1 - 2), jnp.float32)
    t1_ref[0] = jnp.concatenate([gc_i, sr_i, zf], axis=1)
    t1_ref[1] = jnp.concatenate([gc_o, sr_o, zf], axis=1)
    g2 = _dot_t(gb, wrel_ref[...])
    gc2_i = _dot_t(g2, w2i_ref[...][:, 2 * H1:])
    gc2_o = _dot_t(g2, w2o_ref[...][:, 2 * H1:])
    sr2_i = jnp.sum(gc2_i * av2i, axis=1, keepdims=True)
    sr2_o = jnp.sum(gc2_o * av2o, axis=1, keepdims=True)
    t2_ref[0] = jnp.concatenate([gc2_i, sr2_i, sr2_i, zf], axis=1)
    t2_ref[1] = jnp.concatenate([gc2_o, sr2_o, sr2_o, zf], axis=1)


def _headmix(sblk, foldblk):
    feat = sblk[:, :H1]
    z0 = sblk[:, 128:129]
    z1 = sblk[:, 129:130]
    zr = jnp.concatenate([jnp.broadcast_to(z0, (BN, 64)),
                          jnp.broadcast_to(z1, (BN, 64))], axis=1)
    h = jnp.where(z0 > 0, foldblk + feat / jnp.maximum(zr, 1e-30), 0.0)
    return _l2norm(_elu(h))


def _merge(hi_in, ho_in, wmi, bmi, wmo, bmo, wml, bml):
    hi = _dot_t(hi_in, wmi) + bmi
    ho = _dot_t(ho_in, wmo) + bmo
    lam = jax.nn.sigmoid(_dot_t(hi, wml[:, :H1]) + _dot_t(ho, wml[:, H1:])
                         + bml)
    return lam * hi + (1.0 - lam) * ho


def _tc_mid(s1_ref, fold1_ref, wmi_ref, bmi_ref, wmo_ref, bmo_ref,
            wml_ref, bml_ref, w2i_ref, av2i_ref, w2o_ref, av2o_ref,
            feat2_ref, scal2_ref, fold2_ref):
    h_in = _headmix(s1_ref[0], fold1_ref[0])
    h_out = _headmix(s1_ref[1], fold1_ref[1])
    h = _merge(h_in, h_out, wmi_ref[...], bmi_ref[...], wmo_ref[...],
               bmo_ref[...], wml_ref[...], bml_ref[...])
    av2i = av2i_ref[...].reshape(H1)
    av2o = av2o_ref[...].reshape(H1)
    w2i = w2i_ref[...]
    w2o = w2o_ref[...]
    ha_i = _dot_t(h, w2i[:, :H1])
    hb_i = _dot_t(h, w2i[:, H1:2 * H1])
    ha_o = _dot_t(h, w2o[:, :H1])
    hb_o = _dot_t(h, w2o[:, H1:2 * H1])
    sa_i = jnp.sum(ha_i * av2i, axis=1, keepdims=True)
    sb_i = jnp.sum(hb_i * av2i, axis=1, keepdims=True)
    sa_o = jnp.sum(ha_o * av2o, axis=1, keepdims=True)
    sb_o = jnp.sum(hb_o * av2o, axis=1, keepdims=True)
    zf = jnp.zeros((BN, F - H1 - 2), jnp.float32)
    zs = jnp.zeros((BN, 14), jnp.float32)
    feat2_ref[0] = jnp.concatenate([ha_i, sa_i, sa_i, zf], axis=1)
    feat2_ref[1] = jnp.concatenate([hb_o, sb_o, sb_o, zf], axis=1)
    scal2_ref[0] = jnp.concatenate([sb_i, sb_i, zs], axis=1)
    scal2_ref[1] = jnp.concatenate([sa_o, sa_o, zs], axis=1)
    fold2_ref[0] = hb_i
    fold2_ref[1] = ha_o


def _tc_final(s2_ref, fold2_ref, x_ref, wmi_ref, bmi_ref, wmo_ref, bmo_ref,
              wml_ref, bml_ref, went_ref, out_ref):
    h_in2 = _headmix(s2_ref[0], fold2_ref[0])
    h_out2 = _headmix(s2_ref[1], fold2_ref[1])
    h2 = _merge(h_in2, h_out2, wmi_ref[...], bmi_ref[...], wmo_ref[...],
                bmo_ref[...], wml_ref[...], bml_ref[...])
    out_ref[...] = _l2norm(_dot_t(x_ref[...], went_ref[...]) + h2)


def _sc_edge(feat_hbm, scal_hbm, typ_hbm, zero_hbm, eidx_hbm, out_hbm,
             acc, rbuf, cbuf, tbuf, ibuf, *sems):
    semi = sems[0:2]
    semf = sems[2:4]
    semc = sems[4:6]
    semt = sems[6:8]
    c = lax.axis_index("c")
    s = lax.axis_index("s")
    pltpu.sync_copy(zero_hbm.at[pl.ds(s * ROWS_PT, ROWS_PT)],
                    acc.at[pl.ds(s * ROWS_PT, ROWS_PT)])

    base = c * E + s * EPT
    iota16 = lax.broadcasted_iota(jnp.int32, (16,), 0)

    def idx_start(col, sl):
        pltpu.make_async_copy(eidx_hbm.at[:, pl.ds(col, K)], ibuf.at[sl],
                              semi[sl]).start()

    def idx_wait(sl):
        pltpu.make_async_copy(eidx_hbm.at[:, pl.ds(0, K)], ibuf.at[sl],
                              semi[sl]).wait()

    def gathers_start(sl):
        pltpu.make_async_copy(feat_hbm.at[ibuf.at[sl, 0]], rbuf.at[sl],
                              semf[sl]).start()
        pltpu.make_async_copy(scal_hbm.at[ibuf.at[sl, 1]], cbuf.at[sl],
                              semc[sl]).start()
        pltpu.make_async_copy(typ_hbm.at[ibuf.at[sl, 3]], tbuf.at[sl],
                              semt[sl]).start()

    def gathers_wait(sl):
        pltpu.make_async_copy(feat_hbm.at[pl.ds(0, K)], rbuf.at[sl],
                              semf[sl]).wait()
        pltpu.make_async_copy(scal_hbm.at[pl.ds(0, K)], cbuf.at[sl],
                              semc[sl]).wait()
        pltpu.make_async_copy(typ_hbm.at[pl.ds(0, K)], tbuf.at[sl],
                              semt[sl]).wait()

    def compute(sl):
        rb = rbuf.at[sl]
        cb = cbuf.at[sl]
        tb = tbuf.at[sl]

        def edge(e, carry2):
            sv = (rb[e, pl.ds(128, 16)] + cb[e, pl.ds(0, 16)]
                  + tb[e, pl.ds(128, 16)])
            ev = jnp.exp(-jnp.where(sv >= 0, sv, 0.2 * sv))
            ev0 = ev[0]
            ev1 = ev[1]
            for j in range(8):
                evh = ev0 if j < 4 else ev1
                fj = rb[e, pl.ds(j * 16, 16)] + tb[e, pl.ds(j * 16, 16)]
                rb[e, pl.ds(j * 16, 16)] = evh * fj
            rb[e, pl.ds(128, 16)] = jnp.where(iota16 < 2, ev, 0.0)
            return carry2

        lax.fori_loop(0, K, edge, 0, unroll=4)

    pltpu.sync_copy(eidx_hbm.at[:, pl.ds(base, K)], ibuf.at[0])
    gathers_start(0)
    idx_start(base + K, 1)
    plsc.subcore_barrier()

    def pair(ph, carry):
        for sl in range(2):
            i = ph * 2 + sl
            nxt = 1 - sl

            @pl.when(i + 1 < NCH)
            def _():
                idx_wait(nxt)
                gathers_start(nxt)

            gathers_wait(sl)
            compute(sl)
            pltpu.sync_copy(rbuf.at[sl], acc.at[ibuf.at[sl, 2]], add=True)

            @pl.when(i + 2 < NCH)
            def _():
                idx_start(base + (i + 2) * K, sl)
        return carry

    lax.fori_loop(0, NCH // 2, pair, 0)
    plsc.subcore_barrier()
    pltpu.sync_copy(acc.at[pl.ds(s * ROWS_PT, ROWS_PT)],
                    out_hbm.at[c, pl.ds(s * ROWS_PT, ROWS_PT)])


def _edge_pass(feat, scal, typ, zeros_nf, eidx):
    mesh = plsc.VectorSubcoreMesh(core_axis_name="c", subcore_axis_name="s")
    f = pl.kernel(
        _sc_edge,
        out_type=jax.ShapeDtypeStruct((2, N_PAD, F), jnp.float32),
        mesh=mesh,
        compiler_params=pltpu.CompilerParams(use_tc_tiling_on_sc=False),
        scratch_types=[
            pltpu.VMEM_SHARED((N_PAD, F), jnp.float32),
            pltpu.VMEM((2, K, F), jnp.float32),
            pltpu.VMEM((2, K, 16), jnp.float32),
            pltpu.VMEM((2, K, F), jnp.float32),
            pltpu.VMEM((2, 4, K), jnp.int32),
        ] + [pltpu.SemaphoreType.DMA] * 8,
    )
    return f(feat, scal, typ, zeros_nf, eidx)


def kernel(x, g, edge_idx, edge_type, path_idx, path_type, use_path,
           W_fc1_in1, a_in1, W_fc1_out1, a_out1,
           Wm1_in, bm1_in, Wm1_out, bm1_out, Wm1_l, bm1_l,
           W_rel, W_fc2_in, a_in2, W_fc2_out, a_out2,
           Wm2_in, bm2_in, Wm2_out, bm2_out, Wm2_l, bm2_l, W_ent):
    row = edge_idx[0].astype(jnp.int32)
    col = edge_idx[1].astype(jnp.int32)
    et = edge_type.astype(jnp.int32)
    esrc = jnp.concatenate([row, col + N])
    edstg = jnp.concatenate([col, row + N])
    edsts = jnp.concatenate([col, row])
    etyp = jnp.concatenate([et, et + MP])
    eidx = jnp.stack([esrc, edstg, edsts, etyp])
    zeros_nf = jnp.zeros((N_PAD, F), jnp.float32)
    g_p = jnp.pad(g, ((0, MP - M), (0, 0)))
    av1i = a_in1.reshape(1, H1)
    av1o = a_out1.reshape(1, H1)
    av2i = a_in2.reshape(1, H1)
    av2o = a_out2.reshape(1, H1)

    wspec = pl.BlockSpec((H1, 3 * DX), lambda i: (0, 0))
    w1spec = pl.BlockSpec((H1, H1), lambda i: (0, 0))
    avspec = pl.BlockSpec((1, H1), lambda i: (0, 0))
    b1spec = pl.BlockSpec((1, H1), lambda i: (0, 0))
    bl_spec = pl.BlockSpec((1, 1), lambda i: (0, 0))
    wlspec = pl.BlockSpec((1, 2 * H1), lambda i: (0, 0))
    feat_spec = pl.BlockSpec((2, BN, F), lambda i: (0, i, 0))
    scal_spec = pl.BlockSpec((2, BN, 16), lambda i: (0, i, 0))
    fold_spec = pl.BlockSpec((2, BN, H1), lambda i: (0, i, 0))
    x_spec = pl.BlockSpec((BN, DX), lambda i: (i, 0))

    feat1, scal1, fold1 = pl.pallas_call(
        _tc_tables1,
        grid=(NB,),
        in_specs=[x_spec, wspec, avspec, wspec, avspec],
        out_specs=[feat_spec, scal_spec, fold_spec],
        out_shape=[
            jax.ShapeDtypeStruct((2, N, F), jnp.float32),
            jax.ShapeDtypeStruct((2, N, 16), jnp.float32),
            jax.ShapeDtypeStruct((2, N, H1), jnp.float32),
        ],
    )(x, W_fc1_in1, av1i, W_fc1_out1, av1o)

    typ1, typ2 = pl.pallas_call(
        _tc_typetables,
        grid=(1,),
        in_specs=[pl.BlockSpec((MP, DX), lambda i: (0, 0)),
                  wspec, avspec, wspec, avspec,
                  pl.BlockSpec((H1, DX), lambda i: (0, 0)),
                  wspec, avspec, wspec, avspec],
        out_specs=[pl.BlockSpec((2, MP, F), lambda i: (0, 0, 0)),
                   pl.BlockSpec((2, MP, F), lambda i: (0, 0, 0))],
        out_shape=[
            jax.ShapeDtypeStruct((2, MP, F), jnp.float32),
            jax.ShapeDtypeStruct((2, MP, F), jnp.float32),
        ],
    )(g_p, W_fc1_in1, av1i, W_fc1_out1, av1o, W_rel,
      W_fc2_in, av2i, W_fc2_out, av2o)

    s1 = _edge_pass(feat1.reshape(2 * N, F), scal1.reshape(2 * N, 16),
                    typ1.reshape(2 * MP, F), zeros_nf, eidx)

    feat2, scal2, fold2 = pl.pallas_call(
        _tc_mid,
        grid=(NB,),
        in_specs=[feat_spec, fold_spec,
                  w1spec, b1spec, w1spec, b1spec, wlspec, bl_spec,
                  wspec, avspec, wspec, avspec],
        out_specs=[feat_spec, scal_spec, fold_spec],
        out_shape=[
            jax.ShapeDtypeStruct((2, N, F), jnp.float32),
            jax.ShapeDtypeStruct((2, N, 16), jnp.float32),
            jax.ShapeDtypeStruct((2, N, H1), jnp.float32),
        ],
    )(s1, fold1, Wm1_in, bm1_in.reshape(1, H1), Wm1_out,
      bm1_out.reshape(1, H1), Wm1_l, bm1_l.reshape(1, 1),
      W_fc2_in, av2i, W_fc2_out, av2o)

    s2 = _edge_pass(feat2.reshape(2 * N, F), scal2.reshape(2 * N, 16),
                    typ2.reshape(2 * MP, F), zeros_nf, eidx)

    h_prime = pl.pallas_call(
        _tc_final,
        grid=(NB,),
        in_specs=[feat_spec, fold_spec, x_spec,
                  w1spec, b1spec, w1spec, b1spec, wlspec, bl_spec, w1spec],
        out_specs=pl.BlockSpec((BN, H1), lambda i: (i, 0)),
        out_shape=jax.ShapeDtypeStruct((N, H1), jnp.float32),
    )(s2, fold2, x, Wm2_in, bm2_in.reshape(1, H1), Wm2_out,
      bm2_out.reshape(1, H1), Wm2_l, bm2_l.reshape(1, 1), W_ent)

    return h_prime


# speedup vs baseline: 1.5802x; 1.5802x over previous
import functools

import jax
import jax.numpy as jnp
from jax import lax
from jax.experimental import pallas as pl
from jax.experimental.pallas import tpu as pltpu
from jax.experimental.pallas import tpu_sc as plsc

N = 10000
E = 160000
DX = 128
M = 500
MP = 512
H1 = 128
F = 144
NB = 10
BN = N // NB
NTILES = 16
K = 64
NCH = 158
EPT = K * NCH
EP = EPT * NTILES
N_PAD = 10112
ROWS_PT = N_PAD // NTILES


def _dot_t(a, b):
    return lax.dot_general(a, b, (((1,), (1,)), ((), ())),
                           preferred_element_type=jnp.float32)


def _elu(x):
    return jnp.where(x > 0, x, jnp.exp(x) - 1.0)


def _l2norm(x):
    n = jnp.sqrt(jnp.sum(x * x, axis=-1, keepdims=True))
    return x / jnp.maximum(n, 1e-12)


def _tc_tables1(x_ref, wi_ref, avi_ref, wo_ref, avo_ref,
                feat_ref, scal_ref, fold_ref):
    xb = x_ref[...]
    wi = wi_ref[...]
    wo = wo_ref[...]
    avi = avi_ref[...].reshape(H1)
    avo = avo_ref[...].reshape(H1)
    xa_i = _dot_t(xb, wi[:, :DX])
    xb_i = _dot_t(xb, wi[:, DX:2 * DX])
    xa_o = _dot_t(xb, wo[:, :DX])
    xb_o = _dot_t(xb, wo[:, DX:2 * DX])
    sa_i = (xa_i * avi).reshape(-1, 2, 64).sum(-1)
    sb_i = (xb_i * avi).reshape(-1, 2, 64).sum(-1)
    sa_o = (xa_o * avo).reshape(-1, 2, 64).sum(-1)
    sb_o = (xb_o * avo).reshape(-1, 2, 64).sum(-1)
    zf = jnp.zeros((BN, F - H1 - 2), jnp.float32)
    zs = jnp.zeros((BN, 14), jnp.float32)
    feat_ref[0] = jnp.concatenate([xa_i, sa_i, zf], axis=1)
    feat_ref[1] = jnp.concatenate([xb_o, sb_o, zf], axis=1)
    scal_ref[0] = jnp.concatenate([sb_i, zs], axis=1)
    scal_ref[1] = jnp.concatenate([sa_o, zs], axis=1)
    fold_ref[0] = xb_i
    fold_ref[1] = xa_o


def _tc_typetables(g_ref, w1i_ref, av1i_ref, w1o_ref, av1o_ref, wrel_ref,
                   w2i_ref, av2i_ref, w2o_ref, av2o_ref, t1_ref, t2_ref):
    gb = g_ref[...]
    av1i = av1i_ref[...].reshape(H1)
    av1o = av1o_ref[...].reshape(H1)
    av2i = av2i_ref[...].reshape(H1)
    av2o = av2o_ref[...].reshape(H1)
    gc_i = _dot_t(gb, w1i_ref[...][:, 2 * DX:])
    gc_o = _dot_t(gb, w1o_ref[...][:, 2 * DX:])
    sr_i = (gc_i * av1i).reshape(-1, 2, 64).sum(-1)
    sr_o = (gc_o * av1o).reshape(-1, 2, 64).sum(-1)
    zf = jnp.zeros((MP, F - H1 - 2), jnp.float32)
    t1_ref[0] = jnp.concatenate([gc_i, sr_i, zf], axis=1)
    t1_ref[1] = jnp.concatenate([gc_o, sr_o, zf], axis=1)
    g2 = _dot_t(gb, wrel_ref[...])
    gc2_i = _dot_t(g2, w2i_ref[...][:, 2 * H1:])
    gc2_o = _dot_t(g2, w2o_ref[...][:, 2 * H1:])
    sr2_i = jnp.sum(gc2_i * av2i, axis=1, keepdims=True)
    sr2_o = jnp.sum(gc2_o * av2o, axis=1, keepdims=True)
    t2_ref[0] = jnp.concatenate([gc2_i, sr2_i, sr2_i, zf], axis=1)
    t2_ref[1] = jnp.concatenate([gc2_o, sr2_o, sr2_o, zf], axis=1)


def _headmix(sblk, foldblk):
    feat = sblk[:, :H1]
    z0 = sblk[:, 128:129]
    z1 = sblk[:, 129:130]
    zr = jnp.concatenate([jnp.broadcast_to(z0, (BN, 64)),
                          jnp.broadcast_to(z1, (BN, 64))], axis=1)
    h = jnp.where(z0 > 0, foldblk + feat / jnp.maximum(zr, 1e-30), 0.0)
    return _l2norm(_elu(h))


def _merge(hi_in, ho_in, wmi, bmi, wmo, bmo, wml, bml):
    hi = _dot_t(hi_in, wmi) + bmi
    ho = _dot_t(ho_in, wmo) + bmo
    lam = jax.nn.sigmoid(_dot_t(hi, wml[:, :H1]) + _dot_t(ho, wml[:, H1:])
                         + bml)
    return lam * hi + (1.0 - lam) * ho


def _tc_mid(s1_ref, fold1_ref, wmi_ref, bmi_ref, wmo_ref, bmo_ref,
            wml_ref, bml_ref, w2i_ref, av2i_ref, w2o_ref, av2o_ref,
            feat2_ref, scal2_ref, fold2_ref):
    h_in = _headmix(s1_ref[0], fold1_ref[0])
    h_out = _headmix(s1_ref[1], fold1_ref[1])
    h = _merge(h_in, h_out, wmi_ref[...], bmi_ref[...], wmo_ref[...],
               bmo_ref[...], wml_ref[...], bml_ref[...])
    av2i = av2i_ref[...].reshape(H1)
    av2o = av2o_ref[...].reshape(H1)
    w2i = w2i_ref[...]
    w2o = w2o_ref[...]
    ha_i = _dot_t(h, w2i[:, :H1])
    hb_i = _dot_t(h, w2i[:, H1:2 * H1])
    ha_o = _dot_t(h, w2o[:, :H1])
    hb_o = _dot_t(h, w2o[:, H1:2 * H1])
    sa_i = jnp.sum(ha_i * av2i, axis=1, keepdims=True)
    sb_i = jnp.sum(hb_i * av2i, axis=1, keepdims=True)
    sa_o = jnp.sum(ha_o * av2o, axis=1, keepdims=True)
    sb_o = jnp.sum(hb_o * av2o, axis=1, keepdims=True)
    zf = jnp.zeros((BN, F - H1 - 2), jnp.float32)
    zs = jnp.zeros((BN, 14), jnp.float32)
    feat2_ref[0] = jnp.concatenate([ha_i, sa_i, sa_i, zf], axis=1)
    feat2_ref[1] = jnp.concatenate([hb_o, sb_o, sb_o, zf], axis=1)
    scal2_ref[0] = jnp.concatenate([sb_i, sb_i, zs], axis=1)
    scal2_ref[1] = jnp.concatenate([sa_o, sa_o, zs], axis=1)
    fold2_ref[0] = hb_i
    fold2_ref[1] = ha_o


def _tc_final(s2_ref, fold2_ref, x_ref, wmi_ref, bmi_ref, wmo_ref, bmo_ref,
              wml_ref, bml_ref, went_ref, out_ref):
    h_in2 = _headmix(s2_ref[0], fold2_ref[0])
    h_out2 = _headmix(s2_ref[1], fold2_ref[1])
    h2 = _merge(h_in2, h_out2, wmi_ref[...], bmi_ref[...], wmo_ref[...],
                bmo_ref[...], wml_ref[...], bml_ref[...])
    out_ref[...] = _l2norm(_dot_t(x_ref[...], went_ref[...]) + h2)


def _sc_edge(feat_hbm, scal_hbm, typ_hbm, zero_hbm, eidx_hbm, out_hbm,
             acc, rbuf, cbuf, tbuf, ibuf, *sems):
    semi = sems[0:2]
    semf = sems[2:4]
    semc = sems[4:6]
    semt = sems[6:8]
    c = lax.axis_index("c")
    s = lax.axis_index("s")
    pltpu.sync_copy(zero_hbm.at[pl.ds(s * ROWS_PT, ROWS_PT)],
                    acc.at[pl.ds(s * ROWS_PT, ROWS_PT)])

    base = c * EP + s * EPT
    iota16 = lax.broadcasted_iota(jnp.int32, (16,), 0)

    def idx_start(col, sl):
        pltpu.make_async_copy(eidx_hbm.at[:, pl.ds(col, K)], ibuf.at[sl],
                              semi[sl]).start()

    def idx_wait(sl):
        pltpu.make_async_copy(eidx_hbm.at[:, pl.ds(0, K)], ibuf.at[sl],
                              semi[sl]).wait()

    def gathers_start(sl):
        pltpu.make_async_copy(feat_hbm.at[ibuf.at[sl, 0]], rbuf.at[sl],
                              semf[sl]).start()
        pltpu.make_async_copy(scal_hbm.at[ibuf.at[sl, 1]], cbuf.at[sl],
                              semc[sl]).start()
        pltpu.make_async_copy(typ_hbm.at[ibuf.at[sl, 3]], tbuf.at[sl],
                              semt[sl]).start()

    def gathers_wait(sl):
        pltpu.make_async_copy(feat_hbm.at[pl.ds(0, K)], rbuf.at[sl],
                              semf[sl]).wait()
        pltpu.make_async_copy(scal_hbm.at[pl.ds(0, K)], cbuf.at[sl],
                              semc[sl]).wait()
        pltpu.make_async_copy(typ_hbm.at[pl.ds(0, K)], tbuf.at[sl],
                              semt[sl]).wait()

    def compute(sl):
        rb = rbuf.at[sl]
        cb = cbuf.at[sl]
        tb = tbuf.at[sl]

        def edge(e, carry2):
            sv = (rb[e, pl.ds(128, 16)] + cb[e, pl.ds(0, 16)]
                  + tb[e, pl.ds(128, 16)])
            ev = jnp.exp(-jnp.where(sv >= 0, sv, 0.2 * sv))
            ev0 = ev[0]
            ev1 = ev[1]
            for j in range(8):
                evh = ev0 if j < 4 else ev1
                fj = rb[e, pl.ds(j * 16, 16)] + tb[e, pl.ds(j * 16, 16)]
                rb[e, pl.ds(j * 16, 16)] = evh * fj
            rb[e, pl.ds(128, 16)] = jnp.where(iota16 < 2, ev, 0.0)
            return carry2

        lax.fori_loop(0, K, edge, 0)

    pltpu.sync_copy(eidx_hbm.at[:, pl.ds(base, K)], ibuf.at[0])
    gathers_start(0)
    idx_start(base + K, 1)
    plsc.subcore_barrier()

    def pair(ph, carry):
        for sl in range(2):
            i = ph * 2 + sl
            nxt = 1 - sl

            @pl.when(i + 1 < NCH)
            def _():
                idx_wait(nxt)
                gathers_start(nxt)

            gathers_wait(sl)
            compute(sl)
            pltpu.sync_copy(rbuf.at[sl], acc.at[ibuf.at[sl, 2]], add=True)

            @pl.when(i + 2 < NCH)
            def _():
                idx_start(base + (i + 2) * K, sl)
        return carry

    lax.fori_loop(0, NCH // 2, pair, 0)
    plsc.subcore_barrier()
    pltpu.sync_copy(acc.at[pl.ds(s * ROWS_PT, ROWS_PT)],
                    out_hbm.at[c, pl.ds(s * ROWS_PT, ROWS_PT)])


def _edge_pass(feat, scal, typ, zeros_nf, eidx):
    mesh = plsc.VectorSubcoreMesh(core_axis_name="c", subcore_axis_name="s")
    f = pl.kernel(
        _sc_edge,
        out_type=jax.ShapeDtypeStruct((2, N_PAD, F), jnp.float32),
        mesh=mesh,
        compiler_params=pltpu.CompilerParams(use_tc_tiling_on_sc=False),
        scratch_types=[
            pltpu.VMEM_SHARED((N_PAD, F), jnp.float32),
            pltpu.VMEM((2, K, F), jnp.float32),
            pltpu.VMEM((2, K, 16), jnp.float32),
            pltpu.VMEM((2, K, F), jnp.float32),
            pltpu.VMEM((2, 4, K), jnp.int32),
        ] + [pltpu.SemaphoreType.DMA] * 8,
    )
    return f(feat, scal, typ, zeros_nf, eidx)


def kernel(x, g, edge_idx, edge_type, path_idx, path_type, use_path,
           W_fc1_in1, a_in1, W_fc1_out1, a_out1,
           Wm1_in, bm1_in, Wm1_out, bm1_out, Wm1_l, bm1_l,
           W_rel, W_fc2_in, a_in2, W_fc2_out, a_out2,
           Wm2_in, bm2_in, Wm2_out, bm2_out, Wm2_l, bm2_l, W_ent):
    row = edge_idx[0].astype(jnp.int32)
    col = edge_idx[1].astype(jnp.int32)
    et = edge_type.astype(jnp.int32)
    pad = EP - E
    z = jnp.zeros((pad,), jnp.int32)
    zn = jnp.full((pad,), N, jnp.int32)
    esrc = jnp.concatenate([row, z, col + N, zn])
    edstg = jnp.concatenate([col, z, row + N, zn])
    edsts = jnp.concatenate([col, zn, row, zn])
    etyp = jnp.concatenate([et, z, et + MP, z + MP])
    eidx = jnp.stack([esrc, edstg, edsts, etyp])
    zeros_nf = jnp.zeros((N_PAD, F), jnp.float32)
    g_p = jnp.pad(g, ((0, MP - M), (0, 0)))
    av1i = a_in1.reshape(1, H1)
    av1o = a_out1.reshape(1, H1)
    av2i = a_in2.reshape(1, H1)
    av2o = a_out2.reshape(1, H1)

    wspec = pl.BlockSpec((H1, 3 * DX), lambda i: (0, 0))
    w1spec = pl.BlockSpec((H1, H1), lambda i: (0, 0))
    avspec = pl.BlockSpec((1, H1), lambda i: (0, 0))
    b1spec = pl.BlockSpec((1, H1), lambda i: (0, 0))
    bl_spec = pl.BlockSpec((1, 1), lambda i: (0, 0))
    wlspec = pl.BlockSpec((1, 2 * H1), lambda i: (0, 0))
    feat_spec = pl.BlockSpec((2, BN, F), lambda i: (0, i, 0))
    scal_spec = pl.BlockSpec((2, BN, 16), lambda i: (0, i, 0))
    fold_spec = pl.BlockSpec((2, BN, H1), lambda i: (0, i, 0))
    x_spec = pl.BlockSpec((BN, DX), lambda i: (i, 0))

    feat1, scal1, fold1 = pl.pallas_call(
        _tc_tables1,
        grid=(NB,),
        in_specs=[x_spec, wspec, avspec, wspec, avspec],
        out_specs=[feat_spec, scal_spec, fold_spec],
        out_shape=[
            jax.ShapeDtypeStruct((2, N, F), jnp.float32),
            jax.ShapeDtypeStruct((2, N, 16), jnp.float32),
            jax.ShapeDtypeStruct((2, N, H1), jnp.float32),
        ],
    )(x, W_fc1_in1, av1i, W_fc1_out1, av1o)

    typ1, typ2 = pl.pallas_call(
        _tc_typetables,
        grid=(1,),
        in_specs=[pl.BlockSpec((MP, DX), lambda i: (0, 0)),
                  wspec, avspec, wspec, avspec,
                  pl.BlockSpec((H1, DX), lambda i: (0, 0)),
                  wspec, avspec, wspec, avspec],
        out_specs=[pl.BlockSpec((2, MP, F), lambda i: (0, 0, 0)),
                   pl.BlockSpec((2, MP, F), lambda i: (0, 0, 0))],
        out_shape=[
            jax.ShapeDtypeStruct((2, MP, F), jnp.float32),
            jax.ShapeDtypeStruct((2, MP, F), jnp.float32),
        ],
    )(g_p, W_fc1_in1, av1i, W_fc1_out1, av1o, W_rel,
      W_fc2_in, av2i, W_fc2_out, av2o)

    s1 = _edge_pass(feat1.reshape(2 * N, F), scal1.reshape(2 * N, 16),
                    typ1.reshape(2 * MP, F), zeros_nf, eidx)

    feat2, scal2, fold2 = pl.pallas_call(
        _tc_mid,
        grid=(NB,),
        in_specs=[feat_spec, fold_spec,
                  w1spec, b1spec, w1spec, b1spec, wlspec, bl_spec,
                  wspec, avspec, wspec, avspec],
        out_specs=[feat_spec, scal_spec, fold_spec],
        out_shape=[
            jax.ShapeDtypeStruct((2, N, F), jnp.float32),
            jax.ShapeDtypeStruct((2, N, 16), jnp.float32),
            jax.ShapeDtypeStruct((2, N, H1), jnp.float32),
        ],
    )(s1, fold1, Wm1_in, bm1_in.reshape(1, H1), Wm1_out,
      bm1_out.reshape(1, H1), Wm1_l, bm1_l.reshape(1, 1),
      W_fc2_in, av2i, W_fc2_out, av2o)

    s2 = _edge_pass(feat2.reshape(2 * N, F), scal2.reshape(2 * N, 16),
                    typ2.reshape(2 * MP, F), zeros_nf, eidx)

    h_prime = pl.pallas_call(
        _tc_final,
        grid=(NB,),
        in_specs=[feat_spec, fold_spec, x_spec,
                  w1spec, b1spec, w1spec, b1spec, wlspec, bl_spec, w1spec],
        out_specs=pl.BlockSpec((BN, H1), lambda i: (i, 0)),
        out_shape=jax.ShapeDtypeStruct((N, H1), jnp.float32),
    )(s2, fold2, x, Wm2_in, bm2_in.reshape(1, H1), Wm2_out,
      bm2_out.reshape(1, H1), Wm2_l, bm2_l.reshape(1, 1), W_ent)

    return h_prime


# speedup vs baseline: 1.6190x; 1.0245x over previous
import functools

import jax
import jax.numpy as jnp
from jax import lax
from jax.experimental import pallas as pl
from jax.experimental.pallas import tpu as pltpu
from jax.experimental.pallas import tpu_sc as plsc

N = 10000
E = 160000
DX = 128
M = 500
MP = 512
H1 = 128
F = 144
NB = 10
BN = N // NB
NTILES = 16
K = 48
NCH = 210
EPT = K * NCH
EP = EPT * NTILES
N_PAD = 10112
ROWS_PT = N_PAD // NTILES


def _dot_t(a, b):
    return lax.dot_general(a, b, (((1,), (1,)), ((), ())),
                           preferred_element_type=jnp.float32)


def _elu(x):
    return jnp.where(x > 0, x, jnp.exp(x) - 1.0)


def _l2norm(x):
    n = jnp.sqrt(jnp.sum(x * x, axis=-1, keepdims=True))
    return x / jnp.maximum(n, 1e-12)


def _tc_tables1(x_ref, wi_ref, avi_ref, wo_ref, avo_ref,
                feat_ref, scal_ref, fold_ref):
    xb = x_ref[...]
    wi = wi_ref[...]
    wo = wo_ref[...]
    avi = avi_ref[...].reshape(H1)
    avo = avo_ref[...].reshape(H1)
    xa_i = _dot_t(xb, wi[:, :DX])
    xb_i = _dot_t(xb, wi[:, DX:2 * DX])
    xa_o = _dot_t(xb, wo[:, :DX])
    xb_o = _dot_t(xb, wo[:, DX:2 * DX])
    sa_i = (xa_i * avi).reshape(-1, 2, 64).sum(-1)
    sb_i = (xb_i * avi).reshape(-1, 2, 64).sum(-1)
    sa_o = (xa_o * avo).reshape(-1, 2, 64).sum(-1)
    sb_o = (xb_o * avo).reshape(-1, 2, 64).sum(-1)
    zf = jnp.zeros((BN, F - H1 - 2), jnp.float32)
    zs = jnp.zeros((BN, 14), jnp.float32)
    feat_ref[0] = jnp.concatenate([xa_i, sa_i, zf], axis=1)
    feat_ref[1] = jnp.concatenate([xb_o, sb_o, zf], axis=1)
    scal_ref[0] = jnp.concatenate([sb_i, zs], axis=1)
    scal_ref[1] = jnp.concatenate([sa_o, zs], axis=1)
    fold_ref[0] = xb_i
    fold_ref[1] = xa_o


def _tc_typetables(g_ref, w1i_ref, av1i_ref, w1o_ref, av1o_ref, wrel_ref,
                   w2i_ref, av2i_ref, w2o_ref, av2o_ref, t1_ref, t2_ref):
    gb = g_ref[...]
    av1i = av1i_ref[...].reshape(H1)
    av1o = av1o_ref[...].reshape(H1)
    av2i = av2i_ref[...].reshape(H1)
    av2o = av2o_ref[...].reshape(H1)
    gc_i = _dot_t(gb, w1i_ref[...][:, 2 * DX:])
    gc_o = _dot_t(gb, w1o_ref[...][:, 2 * DX:])
    sr_i = (gc_i * av1i).reshape(-1, 2, 64).sum(-1)
    sr_o = (gc_o * av1o).reshape(-1, 2, 64).sum(-1)
    zf = jnp.zeros((MP, F - H1 - 2), jnp.float32)
    t1_ref[0] = jnp.concatenate([gc_i, sr_i, zf], axis=1)
    t1_ref[1] = jnp.concatenate([gc_o, sr_o, zf], axis=1)
    g2 = _dot_t(gb, wrel_ref[...])
    gc2_i = _dot_t(g2, w2i_ref[...][:, 2 * H1:])
    gc2_o = _dot_t(g2, w2o_ref[...][:, 2 * H1:])
    sr2_i = jnp.sum(gc2_i * av2i, axis=1, keepdims=True)
    sr2_o = jnp.sum(gc2_o * av2o, axis=1, keepdims=True)
    t2_ref[0] = jnp.concatenate([gc2_i, sr2_i, sr2_i, zf], axis=1)
    t2_ref[1] = jnp.concatenate([gc2_o, sr2_o, sr2_o, zf], axis=1)


def _headmix(sblk, foldblk):
    feat = sblk[:, :H1]
    z0 = sblk[:, 128:129]
    z1 = sblk[:, 129:130]
    zr = jnp.concatenate([jnp.broadcast_to(z0, (BN, 64)),
                          jnp.broadcast_to(z1, (BN, 64))], axis=1)
    h = jnp.where(z0 > 0, foldblk + feat / jnp.maximum(zr, 1e-30), 0.0)
    return _l2norm(_elu(h))


def _merge(hi_in, ho_in, wmi, bmi, wmo, bmo, wml, bml):
    hi = _dot_t(hi_in, wmi) + bmi
    ho = _dot_t(ho_in, wmo) + bmo
    lam = jax.nn.sigmoid(_dot_t(hi, wml[:, :H1]) + _dot_t(ho, wml[:, H1:])
                         + bml)
    return lam * hi + (1.0 - lam) * ho


def _tc_mid(s1_ref, fold1_ref, wmi_ref, bmi_ref, wmo_ref, bmo_ref,
            wml_ref, bml_ref, w2i_ref, av2i_ref, w2o_ref, av2o_ref,
            feat2_ref, scal2_ref, fold2_ref):
    h_in = _headmix(s1_ref[0], fold1_ref[0])
    h_out = _headmix(s1_ref[1], fold1_ref[1])
    h = _merge(h_in, h_out, wmi_ref[...], bmi_ref[...], wmo_ref[...],
               bmo_ref[...], wml_ref[...], bml_ref[...])
    av2i = av2i_ref[...].reshape(H1)
    av2o = av2o_ref[...].reshape(H1)
    w2i = w2i_ref[...]
    w2o = w2o_ref[...]
    ha_i = _dot_t(h, w2i[:, :H1])
    hb_i = _dot_t(h, w2i[:, H1:2 * H1])
    ha_o = _dot_t(h, w2o[:, :H1])
    hb_o = _dot_t(h, w2o[:, H1:2 * H1])
    sa_i = jnp.sum(ha_i * av2i, axis=1, keepdims=True)
    sb_i = jnp.sum(hb_i * av2i, axis=1, keepdims=True)
    sa_o = jnp.sum(ha_o * av2o, axis=1, keepdims=True)
    sb_o = jnp.sum(hb_o * av2o, axis=1, keepdims=True)
    zf = jnp.zeros((BN, F - H1 - 2), jnp.float32)
    zs = jnp.zeros((BN, 14), jnp.float32)
    feat2_ref[0] = jnp.concatenate([ha_i, sa_i, sa_i, zf], axis=1)
    feat2_ref[1] = jnp.concatenate([hb_o, sb_o, sb_o, zf], axis=1)
    scal2_ref[0] = jnp.concatenate([sb_i, sb_i, zs], axis=1)
    scal2_ref[1] = jnp.concatenate([sa_o, sa_o, zs], axis=1)
    fold2_ref[0] = hb_i
    fold2_ref[1] = ha_o


def _tc_final(s2_ref, fold2_ref, x_ref, wmi_ref, bmi_ref, wmo_ref, bmo_ref,
              wml_ref, bml_ref, went_ref, out_ref):
    h_in2 = _headmix(s2_ref[0], fold2_ref[0])
    h_out2 = _headmix(s2_ref[1], fold2_ref[1])
    h2 = _merge(h_in2, h_out2, wmi_ref[...], bmi_ref[...], wmo_ref[...],
                bmo_ref[...], wml_ref[...], bml_ref[...])
    out_ref[...] = _l2norm(_dot_t(x_ref[...], went_ref[...]) + h2)


def _sc_edge(feat_hbm, scal_hbm, typ_hbm, zero_hbm, eidx_hbm, out_hbm,
             acc, typ_s, rbuf, cbuf, tbuf, ibuf, *sems):
    semi = sems[0:2]
    semf = sems[2:4]
    semc = sems[4:6]
    semt = sems[6:8]
    c = lax.axis_index("c")
    s = lax.axis_index("s")
    pltpu.sync_copy(zero_hbm.at[pl.ds(s * ROWS_PT, ROWS_PT)],
                    acc.at[pl.ds(s * ROWS_PT, ROWS_PT)])
    tpr = MP // NTILES
    pltpu.sync_copy(typ_hbm.at[c, pl.ds(s * tpr, tpr)],
                    typ_s.at[pl.ds(s * tpr, tpr)])
    plsc.subcore_barrier()

    base = c * EP + s * EPT
    iota16 = lax.broadcasted_iota(jnp.int32, (16,), 0)

    def idx_start(col, sl):
        pltpu.make_async_copy(eidx_hbm.at[:, pl.ds(col, K)], ibuf.at[sl],
                              semi[sl]).start()

    def idx_wait(sl):
        pltpu.make_async_copy(eidx_hbm.at[:, pl.ds(0, K)], ibuf.at[sl],
                              semi[sl]).wait()

    def gathers_start(sl):
        pltpu.make_async_copy(feat_hbm.at[ibuf.at[sl, 0]], rbuf.at[sl],
                              semf[sl]).start()
        pltpu.make_async_copy(scal_hbm.at[ibuf.at[sl, 1]], cbuf.at[sl],
                              semc[sl]).start()
        pltpu.make_async_copy(typ_s.at[ibuf.at[sl, 3]], tbuf.at[sl],
                              semt[sl]).start()

    def gathers_wait(sl):
        pltpu.make_async_copy(feat_hbm.at[pl.ds(0, K)], rbuf.at[sl],
                              semf[sl]).wait()
        pltpu.make_async_copy(scal_hbm.at[pl.ds(0, K)], cbuf.at[sl],
                              semc[sl]).wait()
        pltpu.make_async_copy(typ_s.at[pl.ds(0, K)], tbuf.at[sl],
                              semt[sl]).wait()

    def compute(sl):
        rb = rbuf.at[sl]
        cb = cbuf.at[sl]
        tb = tbuf.at[sl]

        def edge(e, carry2):
            sv = (rb[e, pl.ds(128, 16)] + cb[e, pl.ds(0, 16)]
                  + tb[e, pl.ds(128, 16)])
            ev = jnp.exp(-jnp.where(sv >= 0, sv, 0.2 * sv))
            ev0 = ev[0]
            ev1 = ev[1]
            for j in range(8):
                evh = ev0 if j < 4 else ev1
                fj = rb[e, pl.ds(j * 16, 16)] + tb[e, pl.ds(j * 16, 16)]
                rb[e, pl.ds(j * 16, 16)] = evh * fj
            rb[e, pl.ds(128, 16)] = jnp.where(iota16 < 2, ev, 0.0)
            return carry2

        lax.fori_loop(0, K, edge, 0)

    pltpu.sync_copy(eidx_hbm.at[:, pl.ds(base, K)], ibuf.at[0])
    gathers_start(0)
    idx_start(base + K, 1)

    def pair(ph, carry):
        for sl in range(2):
            i = ph * 2 + sl
            nxt = 1 - sl

            @pl.when(i + 1 < NCH)
            def _():
                idx_wait(nxt)
                gathers_start(nxt)

            gathers_wait(sl)
            compute(sl)
            pltpu.sync_copy(rbuf.at[sl], acc.at[ibuf.at[sl, 2]], add=True)

            @pl.when(i + 2 < NCH)
            def _():
                idx_start(base + (i + 2) * K, sl)
        return carry

    lax.fori_loop(0, NCH // 2, pair, 0)
    plsc.subcore_barrier()
    pltpu.sync_copy(acc.at[pl.ds(s * ROWS_PT, ROWS_PT)],
                    out_hbm.at[c, pl.ds(s * ROWS_PT, ROWS_PT)])


def _edge_pass(feat, scal, typ, zeros_nf, eidx):
    mesh = plsc.VectorSubcoreMesh(core_axis_name="c", subcore_axis_name="s")
    f = pl.kernel(
        _sc_edge,
        out_type=jax.ShapeDtypeStruct((2, N_PAD, F), jnp.float32),
        mesh=mesh,
        compiler_params=pltpu.CompilerParams(use_tc_tiling_on_sc=False),
        scratch_types=[
            pltpu.VMEM_SHARED((N_PAD, F), jnp.float32),
            pltpu.VMEM_SHARED((MP, F), jnp.float32),
            pltpu.VMEM((2, K, F), jnp.float32),
            pltpu.VMEM((2, K, 16), jnp.float32),
            pltpu.VMEM((2, K, F), jnp.float32),
            pltpu.VMEM((2, 4, K), jnp.int32),
        ] + [pltpu.SemaphoreType.DMA] * 8,
    )
    return f(feat, scal, typ, zeros_nf, eidx)


def kernel(x, g, edge_idx, edge_type, path_idx, path_type, use_path,
           W_fc1_in1, a_in1, W_fc1_out1, a_out1,
           Wm1_in, bm1_in, Wm1_out, bm1_out, Wm1_l, bm1_l,
           W_rel, W_fc2_in, a_in2, W_fc2_out, a_out2,
           Wm2_in, bm2_in, Wm2_out, bm2_out, Wm2_l, bm2_l, W_ent):
    row = edge_idx[0].astype(jnp.int32)
    col = edge_idx[1].astype(jnp.int32)
    et = edge_type.astype(jnp.int32)
    pad = EP - E
    z = jnp.zeros((pad,), jnp.int32)
    zn = jnp.full((pad,), N, jnp.int32)
    esrc = jnp.concatenate([row, z, col + N, zn])
    edstg = jnp.concatenate([col, z, row + N, zn])
    edsts = jnp.concatenate([col, zn, row, zn])
    etyp = jnp.concatenate([et, z, et, z])
    eidx = jnp.stack([esrc, edstg, edsts, etyp])
    zeros_nf = jnp.zeros((N_PAD, F), jnp.float32)
    g_p = jnp.pad(g, ((0, MP - M), (0, 0)))
    av1i = a_in1.reshape(1, H1)
    av1o = a_out1.reshape(1, H1)
    av2i = a_in2.reshape(1, H1)
    av2o = a_out2.reshape(1, H1)

    wspec = pl.BlockSpec((H1, 3 * DX), lambda i: (0, 0))
    w1spec = pl.BlockSpec((H1, H1), lambda i: (0, 0))
    avspec = pl.BlockSpec((1, H1), lambda i: (0, 0))
    b1spec = pl.BlockSpec((1, H1), lambda i: (0, 0))
    bl_spec = pl.BlockSpec((1, 1), lambda i: (0, 0))
    wlspec = pl.BlockSpec((1, 2 * H1), lambda i: (0, 0))
    feat_spec = pl.BlockSpec((2, BN, F), lambda i: (0, i, 0))
    scal_spec = pl.BlockSpec((2, BN, 16), lambda i: (0, i, 0))
    fold_spec = pl.BlockSpec((2, BN, H1), lambda i: (0, i, 0))
    x_spec = pl.BlockSpec((BN, DX), lambda i: (i, 0))

    feat1, scal1, fold1 = pl.pallas_call(
        _tc_tables1,
        grid=(NB,),
        in_specs=[x_spec, wspec, avspec, wspec, avspec],
        out_specs=[feat_spec, scal_spec, fold_spec],
        out_shape=[
            jax.ShapeDtypeStruct((2, N, F), jnp.float32),
            jax.ShapeDtypeStruct((2, N, 16), jnp.float32),
            jax.ShapeDtypeStruct((2, N, H1), jnp.float32),
        ],
    )(x, W_fc1_in1, av1i, W_fc1_out1, av1o)

    typ1, typ2 = pl.pallas_call(
        _tc_typetables,
        grid=(1,),
        in_specs=[pl.BlockSpec((MP, DX), lambda i: (0, 0)),
                  wspec, avspec, wspec, avspec,
                  pl.BlockSpec((H1, DX), lambda i: (0, 0)),
                  wspec, avspec, wspec, avspec],
        out_specs=[pl.BlockSpec((2, MP, F), lambda i: (0, 0, 0)),
                   pl.BlockSpec((2, MP, F), lambda i: (0, 0, 0))],
        out_shape=[
            jax.ShapeDtypeStruct((2, MP, F), jnp.float32),
            jax.ShapeDtypeStruct((2, MP, F), jnp.float32),
        ],
    )(g_p, W_fc1_in1, av1i, W_fc1_out1, av1o, W_rel,
      W_fc2_in, av2i, W_fc2_out, av2o)

    s1 = _edge_pass(feat1.reshape(2 * N, F), scal1.reshape(2 * N, 16),
                    typ1, zeros_nf, eidx)

    feat2, scal2, fold2 = pl.pallas_call(
        _tc_mid,
        grid=(NB,),
        in_specs=[feat_spec, fold_spec,
                  w1spec, b1spec, w1spec, b1spec, wlspec, bl_spec,
                  wspec, avspec, wspec, avspec],
        out_specs=[feat_spec, scal_spec, fold_spec],
        out_shape=[
            jax.ShapeDtypeStruct((2, N, F), jnp.float32),
            jax.ShapeDtypeStruct((2, N, 16), jnp.float32),
            jax.ShapeDtypeStruct((2, N, H1), jnp.float32),
        ],
    )(s1, fold1, Wm1_in, bm1_in.reshape(1, H1), Wm1_out,
      bm1_out.reshape(1, H1), Wm1_l, bm1_l.reshape(1, 1),
      W_fc2_in, av2i, W_fc2_out, av2o)

    s2 = _edge_pass(feat2.reshape(2 * N, F), scal2.reshape(2 * N, 16),
                    typ2, zeros_nf, eidx)

    h_prime = pl.pallas_call(
        _tc_final,
        grid=(NB,),
        in_specs=[feat_spec, fold_spec, x_spec,
                  w1spec, b1spec, w1spec, b1spec, wlspec, bl_spec, w1spec],
        out_specs=pl.BlockSpec((BN, H1), lambda i: (i, 0)),
        out_shape=jax.ShapeDtypeStruct((N, H1), jnp.float32),
    )(s2, fold2, x, Wm2_in, bm2_in.reshape(1, H1), Wm2_out,
      bm2_out.reshape(1, H1), Wm2_l, bm2_l.reshape(1, 1), W_ent)

    return h_prime


# speedup vs baseline: 1.7373x; 1.0731x over previous
import functools

import jax
import jax.numpy as jnp
from jax import lax
from jax.experimental import pallas as pl
from jax.experimental.pallas import tpu as pltpu
from jax.experimental.pallas import tpu_sc as plsc

N = 10000
E = 160000
DX = 128
M = 500
MP = 512
H1 = 128
F = 144
NB = 10
BN = N // NB
NTILES = 16
K = 40
NCH = 252
EPT = K * NCH
EP = EPT * NTILES
N_PAD = 10112
ROWS_PT = N_PAD // NTILES


def _dot_t(a, b):
    return lax.dot_general(a, b, (((1,), (1,)), ((), ())),
                           preferred_element_type=jnp.float32)


def _elu(x):
    return jnp.where(x > 0, x, jnp.exp(x) - 1.0)


def _l2norm(x):
    n = jnp.sqrt(jnp.sum(x * x, axis=-1, keepdims=True))
    return x / jnp.maximum(n, 1e-12)


def _tc_tables1(x_ref, wi_ref, avi_ref, wo_ref, avo_ref,
                feat_ref, scal_ref, fold_ref):
    xb = x_ref[...]
    wi = wi_ref[...]
    wo = wo_ref[...]
    avi = avi_ref[...].reshape(H1)
    avo = avo_ref[...].reshape(H1)
    xa_i = _dot_t(xb, wi[:, :DX])
    xb_i = _dot_t(xb, wi[:, DX:2 * DX])
    xa_o = _dot_t(xb, wo[:, :DX])
    xb_o = _dot_t(xb, wo[:, DX:2 * DX])
    sa_i = (xa_i * avi).reshape(-1, 2, 64).sum(-1)
    sb_i = (xb_i * avi).reshape(-1, 2, 64).sum(-1)
    sa_o = (xa_o * avo).reshape(-1, 2, 64).sum(-1)
    sb_o = (xb_o * avo).reshape(-1, 2, 64).sum(-1)
    zf = jnp.zeros((BN, F - H1 - 2), jnp.float32)
    zs = jnp.zeros((BN, 14), jnp.float32)
    feat_ref[0] = jnp.concatenate([xa_i, sa_i, zf], axis=1)
    feat_ref[1] = jnp.concatenate([xb_o, sb_o, zf], axis=1)
    scal_ref[0] = jnp.concatenate([sb_i, zs], axis=1)
    scal_ref[1] = jnp.concatenate([sa_o, zs], axis=1)
    fold_ref[0] = xb_i
    fold_ref[1] = xa_o


def _tc_typetables(g_ref, w1i_ref, av1i_ref, w1o_ref, av1o_ref, wrel_ref,
                   w2i_ref, av2i_ref, w2o_ref, av2o_ref, t1_ref, t2_ref):
    gb = g_ref[...]
    av1i = av1i_ref[...].reshape(H1)
    av1o = av1o_ref[...].reshape(H1)
    av2i = av2i_ref[...].reshape(H1)
    av2o = av2o_ref[...].reshape(H1)
    gc_i = _dot_t(gb, w1i_ref[...][:, 2 * DX:])
    gc_o = _dot_t(gb, w1o_ref[...][:, 2 * DX:])
    sr_i = (gc_i * av1i).reshape(-1, 2, 64).sum(-1)
    sr_o = (gc_o * av1o).reshape(-1, 2, 64).sum(-1)
    zf = jnp.zeros((MP, F - H1 - 2), jnp.float32)
    t1_ref[0] = jnp.concatenate([gc_i, sr_i, zf], axis=1)
    t1_ref[1] = jnp.concatenate([gc_o, sr_o, zf], axis=1)
    g2 = _dot_t(gb, wrel_ref[...])
    gc2_i = _dot_t(g2, w2i_ref[...][:, 2 * H1:])
    gc2_o = _dot_t(g2, w2o_ref[...][:, 2 * H1:])
    sr2_i = jnp.sum(gc2_i * av2i, axis=1, keepdims=True)
    sr2_o = jnp.sum(gc2_o * av2o, axis=1, keepdims=True)
    t2_ref[0] = jnp.concatenate([gc2_i, sr2_i, sr2_i, zf], axis=1)
    t2_ref[1] = jnp.concatenate([gc2_o, sr2_o, sr2_o, zf], axis=1)


def _headmix(sblk, foldblk):
    feat = sblk[:, :H1]
    z0 = sblk[:, 128:129]
    z1 = sblk[:, 129:130]
    zr = jnp.concatenate([jnp.broadcast_to(z0, (BN, 64)),
                          jnp.broadcast_to(z1, (BN, 64))], axis=1)
    h = jnp.where(z0 > 0, foldblk + feat / jnp.maximum(zr, 1e-30), 0.0)
    return _l2norm(_elu(h))


def _merge(hi_in, ho_in, wmi, bmi, wmo, bmo, wml, bml):
    hi = _dot_t(hi_in, wmi) + bmi
    ho = _dot_t(ho_in, wmo) + bmo
    lam = jax.nn.sigmoid(_dot_t(hi, wml[:, :H1]) + _dot_t(ho, wml[:, H1:])
                         + bml)
    return lam * hi + (1.0 - lam) * ho


def _tc_mid(s1_ref, fold1_ref, wmi_ref, bmi_ref, wmo_ref, bmo_ref,
            wml_ref, bml_ref, w2i_ref, av2i_ref, w2o_ref, av2o_ref,
            feat2_ref, scal2_ref, fold2_ref):
    h_in = _headmix(s1_ref[0], fold1_ref[0])
    h_out = _headmix(s1_ref[1], fold1_ref[1])
    h = _merge(h_in, h_out, wmi_ref[...], bmi_ref[...], wmo_ref[...],
               bmo_ref[...], wml_ref[...], bml_ref[...])
    av2i = av2i_ref[...].reshape(H1)
    av2o = av2o_ref[...].reshape(H1)
    w2i = w2i_ref[...]
    w2o = w2o_ref[...]
    ha_i = _dot_t(h, w2i[:, :H1])
    hb_i = _dot_t(h, w2i[:, H1:2 * H1])
    ha_o = _dot_t(h, w2o[:, :H1])
    hb_o = _dot_t(h, w2o[:, H1:2 * H1])
    sa_i = jnp.sum(ha_i * av2i, axis=1, keepdims=True)
    sb_i = jnp.sum(hb_i * av2i, axis=1, keepdims=True)
    sa_o = jnp.sum(ha_o * av2o, axis=1, keepdims=True)
    sb_o = jnp.sum(hb_o * av2o, axis=1, keepdims=True)
    zf = jnp.zeros((BN, F - H1 - 2), jnp.float32)
    zs = jnp.zeros((BN, 14), jnp.float32)
    feat2_ref[0] = jnp.concatenate([ha_i, sa_i, sa_i, zf], axis=1)
    feat2_ref[1] = jnp.concatenate([hb_o, sb_o, sb_o, zf], axis=1)
    scal2_ref[0] = jnp.concatenate([sb_i, sb_i, zs], axis=1)
    scal2_ref[1] = jnp.concatenate([sa_o, sa_o, zs], axis=1)
    fold2_ref[0] = hb_i
    fold2_ref[1] = ha_o


def _tc_final(s2_ref, fold2_ref, x_ref, wmi_ref, bmi_ref, wmo_ref, bmo_ref,
              wml_ref, bml_ref, went_ref, out_ref):
    h_in2 = _headmix(s2_ref[0], fold2_ref[0])
    h_out2 = _headmix(s2_ref[1], fold2_ref[1])
    h2 = _merge(h_in2, h_out2, wmi_ref[...], bmi_ref[...], wmo_ref[...],
                bmo_ref[...], wml_ref[...], bml_ref[...])
    out_ref[...] = _l2norm(_dot_t(x_ref[...], went_ref[...]) + h2)


def _sc_edge(feat_hbm, scal_hbm, typ_hbm, zero_hbm, eidx_hbm, out_hbm,
             acc, rbuf, cbuf, tbuf, ibuf, *sems):
    semi = sems[0:4]
    semf = sems[4:7]
    semc = sems[7:10]
    semt = sems[10:13]
    semsc = sems[13:16]
    c = lax.axis_index("c")
    s = lax.axis_index("s")
    pltpu.sync_copy(zero_hbm.at[pl.ds(s * ROWS_PT, ROWS_PT)],
                    acc.at[pl.ds(s * ROWS_PT, ROWS_PT)])

    base = c * EP + s * EPT
    iota16 = lax.broadcasted_iota(jnp.int32, (16,), 0)

    def idx_start(col, m):
        pltpu.make_async_copy(eidx_hbm.at[:, pl.ds(col, K)], ibuf.at[m],
                              semi[m]).start()

    def idx_wait(m):
        pltpu.make_async_copy(eidx_hbm.at[:, pl.ds(0, K)], ibuf.at[m],
                              semi[m]).wait()

    def gathers_start(d, m):
        pltpu.make_async_copy(feat_hbm.at[ibuf.at[m, 0]], rbuf.at[d],
                              semf[d]).start()
        pltpu.make_async_copy(scal_hbm.at[ibuf.at[m, 1]], cbuf.at[d],
                              semc[d]).start()
        pltpu.make_async_copy(typ_hbm.at[ibuf.at[m, 3]], tbuf.at[d],
                              semt[d]).start()

    def gathers_wait(d):
        pltpu.make_async_copy(feat_hbm.at[pl.ds(0, K)], rbuf.at[d],
                              semf[d]).wait()
        pltpu.make_async_copy(scal_hbm.at[pl.ds(0, K)], cbuf.at[d],
                              semc[d]).wait()
        pltpu.make_async_copy(typ_hbm.at[pl.ds(0, K)], tbuf.at[d],
                              semt[d]).wait()

    def scatter_start(d, m):
        pltpu.async_copy(rbuf.at[d], acc.at[ibuf.at[m, 2]], semsc[d],
                         add=True)

    def scatter_wait(d):
        pltpu.make_async_copy(rbuf.at[d], acc.at[ibuf.at[0, 2]],
                              semsc[d]).wait()

    def compute(d):
        rb = rbuf.at[d]
        cb = cbuf.at[d]
        tb = tbuf.at[d]

        def edge(e, carry2):
            sv = (rb[e, pl.ds(128, 16)] + cb[e, pl.ds(0, 16)]
                  + tb[e, pl.ds(128, 16)])
            ev = jnp.exp(-jnp.where(sv >= 0, sv, 0.2 * sv))
            ev0 = ev[0]
            ev1 = ev[1]
            for j in range(8):
                evh = ev0 if j < 4 else ev1
                fj = rb[e, pl.ds(j * 16, 16)] + tb[e, pl.ds(j * 16, 16)]
                rb[e, pl.ds(j * 16, 16)] = evh * fj
            rb[e, pl.ds(128, 16)] = jnp.where(iota16 < 2, ev, 0.0)
            return carry2

        lax.fori_loop(0, K, edge, 0)

    pltpu.sync_copy(eidx_hbm.at[:, pl.ds(base, K)], ibuf.at[0])
    gathers_start(0, 0)
    idx_start(base + K, 1)
    plsc.subcore_barrier()

    def block(ph, carry):
        for j2 in range(12):
            i = ph * 12 + j2
            d = j2 % 3
            m = j2 % 4
            d1 = (j2 + 1) % 3
            m1 = (j2 + 1) % 4
            m2 = (j2 + 2) % 4

            @pl.when(i >= 2)
            def _():
                scatter_wait(d1)

            @pl.when(i + 1 < NCH)
            def _():
                idx_wait(m1)
                gathers_start(d1, m1)

            gathers_wait(d)
            compute(d)
            scatter_start(d, m)

            @pl.when(i + 2 < NCH)
            def _():
                idx_start(base + (i + 2) * K, m2)
        return carry

    lax.fori_loop(0, NCH // 12, block, 0)
    scatter_wait((NCH - 2) % 3)
    scatter_wait((NCH - 1) % 3)
    plsc.subcore_barrier()
    pltpu.sync_copy(acc.at[pl.ds(s * ROWS_PT, ROWS_PT)],
                    out_hbm.at[c, pl.ds(s * ROWS_PT, ROWS_PT)])


def _edge_pass(feat, scal, typ, zeros_nf, eidx):
    mesh = plsc.VectorSubcoreMesh(core_axis_name="c", subcore_axis_name="s")
    f = pl.kernel(
        _sc_edge,
        out_type=jax.ShapeDtypeStruct((2, N_PAD, F), jnp.float32),
        mesh=mesh,
        compiler_params=pltpu.CompilerParams(use_tc_tiling_on_sc=False),
        scratch_types=[
            pltpu.VMEM_SHARED((N_PAD, F), jnp.float32),
            pltpu.VMEM((3, K, F), jnp.float32),
            pltpu.VMEM((3, K, 16), jnp.float32),
            pltpu.VMEM((3, K, F), jnp.float32),
            pltpu.VMEM((4, 4, K), jnp.int32),
        ] + [pltpu.SemaphoreType.DMA] * 16,
    )
    return f(feat, scal, typ, zeros_nf, eidx)


def kernel(x, g, edge_idx, edge_type, path_idx, path_type, use_path,
           W_fc1_in1, a_in1, W_fc1_out1, a_out1,
           Wm1_in, bm1_in, Wm1_out, bm1_out, Wm1_l, bm1_l,
           W_rel, W_fc2_in, a_in2, W_fc2_out, a_out2,
           Wm2_in, bm2_in, Wm2_out, bm2_out, Wm2_l, bm2_l, W_ent):
    row = edge_idx[0].astype(jnp.int32)
    col = edge_idx[1].astype(jnp.int32)
    et = edge_type.astype(jnp.int32)
    pad = EP - E
    z = jnp.zeros((pad,), jnp.int32)
    zn = jnp.full((pad,), N, jnp.int32)
    esrc = jnp.concatenate([row, z, col + N, zn])
    edstg = jnp.concatenate([col, z, row + N, zn])
    edsts = jnp.concatenate([col, zn, row, zn])
    etyp = jnp.concatenate([et, z, et + MP, z + MP])
    eidx = jnp.stack([esrc, edstg, edsts, etyp])
    zeros_nf = jnp.zeros((N_PAD, F), jnp.float32)
    g_p = jnp.pad(g, ((0, MP - M), (0, 0)))
    av1i = a_in1.reshape(1, H1)
    av1o = a_out1.reshape(1, H1)
    av2i = a_in2.reshape(1, H1)
    av2o = a_out2.reshape(1, H1)

    wspec = pl.BlockSpec((H1, 3 * DX), lambda i: (0, 0))
    w1spec = pl.BlockSpec((H1, H1), lambda i: (0, 0))
    avspec = pl.BlockSpec((1, H1), lambda i: (0, 0))
    b1spec = pl.BlockSpec((1, H1), lambda i: (0, 0))
    bl_spec = pl.BlockSpec((1, 1), lambda i: (0, 0))
    wlspec = pl.BlockSpec((1, 2 * H1), lambda i: (0, 0))
    feat_spec = pl.BlockSpec((2, BN, F), lambda i: (0, i, 0))
    scal_spec = pl.BlockSpec((2, BN, 16), lambda i: (0, i, 0))
    fold_spec = pl.BlockSpec((2, BN, H1), lambda i: (0, i, 0))
    x_spec = pl.BlockSpec((BN, DX), lambda i: (i, 0))

    feat1, scal1, fold1 = pl.pallas_call(
        _tc_tables1,
        grid=(NB,),
        in_specs=[x_spec, wspec, avspec, wspec, avspec],
        out_specs=[feat_spec, scal_spec, fold_spec],
        out_shape=[
            jax.ShapeDtypeStruct((2, N, F), jnp.float32),
            jax.ShapeDtypeStruct((2, N, 16), jnp.float32),
            jax.ShapeDtypeStruct((2, N, H1), jnp.float32),
        ],
    )(x, W_fc1_in1, av1i, W_fc1_out1, av1o)

    typ1, typ2 = pl.pallas_call(
        _tc_typetables,
        grid=(1,),
        in_specs=[pl.BlockSpec((MP, DX), lambda i: (0, 0)),
                  wspec, avspec, wspec, avspec,
                  pl.BlockSpec((H1, DX), lambda i: (0, 0)),
                  wspec, avspec, wspec, avspec],
        out_specs=[pl.BlockSpec((2, MP, F), lambda i: (0, 0, 0)),
                   pl.BlockSpec((2, MP, F), lambda i: (0, 0, 0))],
        out_shape=[
            jax.ShapeDtypeStruct((2, MP, F), jnp.float32),
            jax.ShapeDtypeStruct((2, MP, F), jnp.float32),
        ],
    )(g_p, W_fc1_in1, av1i, W_fc1_out1, av1o, W_rel,
      W_fc2_in, av2i, W_fc2_out, av2o)

    s1 = _edge_pass(feat1.reshape(2 * N, F), scal1.reshape(2 * N, 16),
                    typ1.reshape(2 * MP, F), zeros_nf, eidx)

    feat2, scal2, fold2 = pl.pallas_call(
        _tc_mid,
        grid=(NB,),
        in_specs=[feat_spec, fold_spec,
                  w1spec, b1spec, w1spec, b1spec, wlspec, bl_spec,
                  wspec, avspec, wspec, avspec],
        out_specs=[feat_spec, scal_spec, fold_spec],
        out_shape=[
            jax.ShapeDtypeStruct((2, N, F), jnp.float32),
            jax.ShapeDtypeStruct((2, N, 16), jnp.float32),
            jax.ShapeDtypeStruct((2, N, H1), jnp.float32),
        ],
    )(s1, fold1, Wm1_in, bm1_in.reshape(1, H1), Wm1_out,
      bm1_out.reshape(1, H1), Wm1_l, bm1_l.reshape(1, 1),
      W_fc2_in, av2i, W_fc2_out, av2o)

    s2 = _edge_pass(feat2.reshape(2 * N, F), scal2.reshape(2 * N, 16),
                    typ2.reshape(2 * MP, F), zeros_nf, eidx)

    h_prime = pl.pallas_call(
        _tc_final,
        grid=(NB,),
        in_specs=[feat_spec, fold_spec, x_spec,
                  w1spec, b1spec, w1spec, b1spec, wlspec, bl_spec, w1spec],
        out_specs=pl.BlockSpec((BN, H1), lambda i: (i, 0)),
        out_shape=jax.ShapeDtypeStruct((N, H1), jnp.float32),
    )(s2, fold2, x, Wm2_in, bm2_in.reshape(1, H1), Wm2_out,
      bm2_out.reshape(1, H1), Wm2_l, bm2_l.reshape(1, 1), W_ent)

    return h_prime


# speedup vs baseline: 2.0109x; 1.1575x over previous
import functools

import jax
import jax.numpy as jnp
from jax import lax
from jax.experimental import pallas as pl
from jax.experimental.pallas import tpu as pltpu
from jax.experimental.pallas import tpu_sc as plsc

N = 10000
E = 160000
DX = 128
M = 500
MP = 512
H1 = 128
F = 144
NB = 10
BN = N // NB
NTILES = 16
K = 40
NCH = 252
EPT = K * NCH
EP = EPT * NTILES
N_PAD = 10112
ROWS_PT = N_PAD // NTILES


def _dot_t(a, b):
    return lax.dot_general(a, b, (((1,), (1,)), ((), ())),
                           preferred_element_type=jnp.float32)


def _elu(x):
    return jnp.where(x > 0, x, jnp.exp(x) - 1.0)


def _l2norm(x):
    n = jnp.sqrt(jnp.sum(x * x, axis=-1, keepdims=True))
    return x / jnp.maximum(n, 1e-12)


def _tc_tables1(x_ref, wi_ref, avi_ref, wo_ref, avo_ref,
                feat_ref, scal_ref, fold_ref):
    xb = x_ref[...]
    wi = wi_ref[...]
    wo = wo_ref[...]
    avi = avi_ref[...].reshape(H1)
    avo = avo_ref[...].reshape(H1)
    xa_i = _dot_t(xb, wi[:, :DX])
    xb_i = _dot_t(xb, wi[:, DX:2 * DX])
    xa_o = _dot_t(xb, wo[:, :DX])
    xb_o = _dot_t(xb, wo[:, DX:2 * DX])
    sa_i = (xa_i * avi).reshape(-1, 2, 64).sum(-1)
    sb_i = (xb_i * avi).reshape(-1, 2, 64).sum(-1)
    sa_o = (xa_o * avo).reshape(-1, 2, 64).sum(-1)
    sb_o = (xb_o * avo).reshape(-1, 2, 64).sum(-1)
    zf = jnp.zeros((BN, F - H1 - 2), jnp.float32)
    zs = jnp.zeros((BN, 14), jnp.float32)
    feat_ref[0] = jnp.concatenate([xa_i, sa_i, zf], axis=1)
    feat_ref[1] = jnp.concatenate([xb_o, sb_o, zf], axis=1)
    scal_ref[0] = jnp.concatenate([sb_i, zs], axis=1)
    scal_ref[1] = jnp.concatenate([sa_o, zs], axis=1)
    fold_ref[0] = xb_i
    fold_ref[1] = xa_o


def _tc_typetables(g_ref, w1i_ref, av1i_ref, w1o_ref, av1o_ref, wrel_ref,
                   w2i_ref, av2i_ref, w2o_ref, av2o_ref, t1_ref, t2_ref):
    gb = g_ref[...]
    av1i = av1i_ref[...].reshape(H1)
    av1o = av1o_ref[...].reshape(H1)
    av2i = av2i_ref[...].reshape(H1)
    av2o = av2o_ref[...].reshape(H1)
    gc_i = _dot_t(gb, w1i_ref[...][:, 2 * DX:])
    gc_o = _dot_t(gb, w1o_ref[...][:, 2 * DX:])
    sr_i = (gc_i * av1i).reshape(-1, 2, 64).sum(-1)
    sr_o = (gc_o * av1o).reshape(-1, 2, 64).sum(-1)
    zf = jnp.zeros((MP, F - H1 - 2), jnp.float32)
    t1_ref[0] = jnp.concatenate([gc_i, sr_i, zf], axis=1)
    t1_ref[1] = jnp.concatenate([gc_o, sr_o, zf], axis=1)
    g2 = _dot_t(gb, wrel_ref[...])
    gc2_i = _dot_t(g2, w2i_ref[...][:, 2 * H1:])
    gc2_o = _dot_t(g2, w2o_ref[...][:, 2 * H1:])
    sr2_i = jnp.sum(gc2_i * av2i, axis=1, keepdims=True)
    sr2_o = jnp.sum(gc2_o * av2o, axis=1, keepdims=True)
    t2_ref[0] = jnp.concatenate([gc2_i, sr2_i, sr2_i, zf], axis=1)
    t2_ref[1] = jnp.concatenate([gc2_o, sr2_o, sr2_o, zf], axis=1)


def _headmix(sblk, foldblk):
    feat = sblk[:, :H1]
    z0 = sblk[:, 128:129]
    z1 = sblk[:, 129:130]
    zr = jnp.concatenate([jnp.broadcast_to(z0, (BN, 64)),
                          jnp.broadcast_to(z1, (BN, 64))], axis=1)
    h = jnp.where(z0 > 0, foldblk + feat / jnp.maximum(zr, 1e-30), 0.0)
    return _l2norm(_elu(h))


def _merge(hi_in, ho_in, wmi, bmi, wmo, bmo, wml, bml):
    hi = _dot_t(hi_in, wmi) + bmi
    ho = _dot_t(ho_in, wmo) + bmo
    lam = jax.nn.sigmoid(_dot_t(hi, wml[:, :H1]) + _dot_t(ho, wml[:, H1:])
                         + bml)
    return lam * hi + (1.0 - lam) * ho


def _tc_mid(s1_ref, fold1_ref, wmi_ref, bmi_ref, wmo_ref, bmo_ref,
            wml_ref, bml_ref, w2i_ref, av2i_ref, w2o_ref, av2o_ref,
            feat2_ref, scal2_ref, fold2_ref):
    h_in = _headmix(s1_ref[0], fold1_ref[0])
    h_out = _headmix(s1_ref[1], fold1_ref[1])
    h = _merge(h_in, h_out, wmi_ref[...], bmi_ref[...], wmo_ref[...],
               bmo_ref[...], wml_ref[...], bml_ref[...])
    av2i = av2i_ref[...].reshape(H1)
    av2o = av2o_ref[...].reshape(H1)
    w2i = w2i_ref[...]
    w2o = w2o_ref[...]
    ha_i = _dot_t(h, w2i[:, :H1])
    hb_i = _dot_t(h, w2i[:, H1:2 * H1])
    ha_o = _dot_t(h, w2o[:, :H1])
    hb_o = _dot_t(h, w2o[:, H1:2 * H1])
    sa_i = jnp.sum(ha_i * av2i, axis=1, keepdims=True)
    sb_i = jnp.sum(hb_i * av2i, axis=1, keepdims=True)
    sa_o = jnp.sum(ha_o * av2o, axis=1, keepdims=True)
    sb_o = jnp.sum(hb_o * av2o, axis=1, keepdims=True)
    zf = jnp.zeros((BN, F - H1 - 2), jnp.float32)
    zs = jnp.zeros((BN, 14), jnp.float32)
    feat2_ref[0] = jnp.concatenate([ha_i, sa_i, sa_i, zf], axis=1)
    feat2_ref[1] = jnp.concatenate([hb_o, sb_o, sb_o, zf], axis=1)
    scal2_ref[0] = jnp.concatenate([sb_i, sb_i, zs], axis=1)
    scal2_ref[1] = jnp.concatenate([sa_o, sa_o, zs], axis=1)
    fold2_ref[0] = hb_i
    fold2_ref[1] = ha_o


def _tc_final(s2_ref, fold2_ref, x_ref, wmi_ref, bmi_ref, wmo_ref, bmo_ref,
              wml_ref, bml_ref, went_ref, out_ref):
    h_in2 = _headmix(s2_ref[0], fold2_ref[0])
    h_out2 = _headmix(s2_ref[1], fold2_ref[1])
    h2 = _merge(h_in2, h_out2, wmi_ref[...], bmi_ref[...], wmo_ref[...],
                bmo_ref[...], wml_ref[...], bml_ref[...])
    out_ref[...] = _l2norm(_dot_t(x_ref[...], went_ref[...]) + h2)


def _sc_edge(feat_hbm, scal_hbm, typ_hbm, zero_hbm, eidx_hbm, out_hbm,
             acc, rbuf, cbuf, tbuf, ibuf, *sems):
    semi = sems[0:4]
    semf = sems[4:7]
    semc = sems[7:10]
    semt = sems[10:13]
    semsc = sems[13:16]
    c = lax.axis_index("c")
    s = lax.axis_index("s")
    pltpu.sync_copy(zero_hbm.at[pl.ds(s * ROWS_PT, ROWS_PT)],
                    acc.at[pl.ds(s * ROWS_PT, ROWS_PT)])

    base = c * EP + s * EPT
    iota16 = lax.broadcasted_iota(jnp.int32, (16,), 0)

    def idx_start(col, m):
        pltpu.make_async_copy(eidx_hbm.at[:, pl.ds(col, K)], ibuf.at[m],
                              semi[m]).start()

    def idx_wait(m):
        pltpu.make_async_copy(eidx_hbm.at[:, pl.ds(0, K)], ibuf.at[m],
                              semi[m]).wait()

    def gathers_start(d, m):
        pltpu.make_async_copy(feat_hbm.at[ibuf.at[m, 0]], rbuf.at[d],
                              semf[d]).start()
        pltpu.make_async_copy(scal_hbm.at[ibuf.at[m, 1]], cbuf.at[d],
                              semc[d]).start()
        pltpu.make_async_copy(typ_hbm.at[ibuf.at[m, 3]], tbuf.at[d],
                              semt[d]).start()

    def gathers_wait(d):
        pltpu.make_async_copy(feat_hbm.at[pl.ds(0, K)], rbuf.at[d],
                              semf[d]).wait()
        pltpu.make_async_copy(scal_hbm.at[pl.ds(0, K)], cbuf.at[d],
                              semc[d]).wait()
        pltpu.make_async_copy(typ_hbm.at[pl.ds(0, K)], tbuf.at[d],
                              semt[d]).wait()

    def scatter_start(d, m):
        pltpu.async_copy(rbuf.at[d], acc.at[ibuf.at[m, 2]], semsc[d],
                         add=True)

    def scatter_wait(d):
        pltpu.make_async_copy(rbuf.at[d], acc.at[ibuf.at[0, 2]],
                              semsc[d]).wait()

    def compute(d):
        rb = rbuf.at[d]
        cb = cbuf.at[d]
        tb = tbuf.at[d]

        @plsc.parallel_loop(0, K)
        def edge(e):
            sv = (rb[e, pl.ds(128, 16)] + cb[e, pl.ds(0, 16)]
                  + tb[e, pl.ds(128, 16)])
            ev = jnp.exp(-jnp.where(sv >= 0, sv, 0.2 * sv))
            ev0 = ev[0]
            ev1 = ev[1]
            for j in range(8):
                evh = ev0 if j < 4 else ev1
                fj = rb[e, pl.ds(j * 16, 16)] + tb[e, pl.ds(j * 16, 16)]
                rb[e, pl.ds(j * 16, 16)] = evh * fj
            rb[e, pl.ds(128, 16)] = jnp.where(iota16 < 2, ev, 0.0)

    pltpu.sync_copy(eidx_hbm.at[:, pl.ds(base, K)], ibuf.at[0])
    gathers_start(0, 0)
    idx_start(base + K, 1)
    plsc.subcore_barrier()

    def block(ph, carry):
        for j2 in range(12):
            i = ph * 12 + j2
            d = j2 % 3
            m = j2 % 4
            d1 = (j2 + 1) % 3
            m1 = (j2 + 1) % 4
            m2 = (j2 + 2) % 4

            @pl.when(i >= 2)
            def _():
                scatter_wait(d1)

            @pl.when(i + 1 < NCH)
            def _():
                idx_wait(m1)
                gathers_start(d1, m1)

            gathers_wait(d)
            compute(d)
            scatter_start(d, m)

            @pl.when(i + 2 < NCH)
            def _():
                idx_start(base + (i + 2) * K, m2)
        return carry

    lax.fori_loop(0, NCH // 12, block, 0)
    scatter_wait((NCH - 2) % 3)
    scatter_wait((NCH - 1) % 3)
    plsc.subcore_barrier()
    pltpu.sync_copy(acc.at[pl.ds(s * ROWS_PT, ROWS_PT)],
                    out_hbm.at[c, pl.ds(s * ROWS_PT, ROWS_PT)])


def _edge_pass(feat, scal, typ, zeros_nf, eidx):
    mesh = plsc.VectorSubcoreMesh(core_axis_name="c", subcore_axis_name="s")
    f = pl.kernel(
        _sc_edge,
        out_type=jax.ShapeDtypeStruct((2, N_PAD, F), jnp.float32),
        mesh=mesh,
        compiler_params=pltpu.CompilerParams(use_tc_tiling_on_sc=False),
        scratch_types=[
            pltpu.VMEM_SHARED((N_PAD, F), jnp.float32),
            pltpu.VMEM((3, K, F), jnp.float32),
            pltpu.VMEM((3, K, 16), jnp.float32),
            pltpu.VMEM((3, K, F), jnp.float32),
            pltpu.VMEM((4, 4, K), jnp.int32),
        ] + [pltpu.SemaphoreType.DMA] * 16,
    )
    return f(feat, scal, typ, zeros_nf, eidx)


def kernel(x, g, edge_idx, edge_type, path_idx, path_type, use_path,
           W_fc1_in1, a_in1, W_fc1_out1, a_out1,
           Wm1_in, bm1_in, Wm1_out, bm1_out, Wm1_l, bm1_l,
           W_rel, W_fc2_in, a_in2, W_fc2_out, a_out2,
           Wm2_in, bm2_in, Wm2_out, bm2_out, Wm2_l, bm2_l, W_ent):
    row = edge_idx[0].astype(jnp.int32)
    col = edge_idx[1].astype(jnp.int32)
    et = edge_type.astype(jnp.int32)
    pad = EP - E
    z = jnp.zeros((pad,), jnp.int32)
    zn = jnp.full((pad,), N, jnp.int32)
    esrc = jnp.concatenate([row, z, col + N, zn])
    edstg = jnp.concatenate([col, z, row + N, zn])
    edsts = jnp.concatenate([col, zn, row, zn])
    etyp = jnp.concatenate([et, z, et + MP, z + MP])
    eidx = jnp.stack([esrc, edstg, edsts, etyp])
    zeros_nf = jnp.zeros((N_PAD, F), jnp.float32)
    g_p = jnp.pad(g, ((0, MP - M), (0, 0)))
    av1i = a_in1.reshape(1, H1)
    av1o = a_out1.reshape(1, H1)
    av2i = a_in2.reshape(1, H1)
    av2o = a_out2.reshape(1, H1)

    wspec = pl.BlockSpec((H1, 3 * DX), lambda i: (0, 0))
    w1spec = pl.BlockSpec((H1, H1), lambda i: (0, 0))
    avspec = pl.BlockSpec((1, H1), lambda i: (0, 0))
    b1spec = pl.BlockSpec((1, H1), lambda i: (0, 0))
    bl_spec = pl.BlockSpec((1, 1), lambda i: (0, 0))
    wlspec = pl.BlockSpec((1, 2 * H1), lambda i: (0, 0))
    feat_spec = pl.BlockSpec((2, BN, F), lambda i: (0, i, 0))
    scal_spec = pl.BlockSpec((2, BN, 16), lambda i: (0, i, 0))
    fold_spec = pl.BlockSpec((2, BN, H1), lambda i: (0, i, 0))
    x_spec = pl.BlockSpec((BN, DX), lambda i: (i, 0))

    feat1, scal1, fold1 = pl.pallas_call(
        _tc_tables1,
        grid=(NB,),
        in_specs=[x_spec, wspec, avspec, wspec, avspec],
        out_specs=[feat_spec, scal_spec, fold_spec],
        out_shape=[
            jax.ShapeDtypeStruct((2, N, F), jnp.float32),
            jax.ShapeDtypeStruct((2, N, 16), jnp.float32),
            jax.ShapeDtypeStruct((2, N, H1), jnp.float32),
        ],
    )(x, W_fc1_in1, av1i, W_fc1_out1, av1o)

    typ1, typ2 = pl.pallas_call(
        _tc_typetables,
        grid=(1,),
        in_specs=[pl.BlockSpec((MP, DX), lambda i: (0, 0)),
                  wspec, avspec, wspec, avspec,
                  pl.BlockSpec((H1, DX), lambda i: (0, 0)),
                  wspec, avspec, wspec, avspec],
        out_specs=[pl.BlockSpec((2, MP, F), lambda i: (0, 0, 0)),
                   pl.BlockSpec((2, MP, F), lambda i: (0, 0, 0))],
        out_shape=[
            jax.ShapeDtypeStruct((2, MP, F), jnp.float32),
            jax.ShapeDtypeStruct((2, MP, F), jnp.float32),
        ],
    )(g_p, W_fc1_in1, av1i, W_fc1_out1, av1o, W_rel,
      W_fc2_in, av2i, W_fc2_out, av2o)

    s1 = _edge_pass(feat1.reshape(2 * N, F), scal1.reshape(2 * N, 16),
                    typ1.reshape(2 * MP, F), zeros_nf, eidx)

    feat2, scal2, fold2 = pl.pallas_call(
        _tc_mid,
        grid=(NB,),
        in_specs=[feat_spec, fold_spec,
                  w1spec, b1spec, w1spec, b1spec, wlspec, bl_spec,
                  wspec, avspec, wspec, avspec],
        out_specs=[feat_spec, scal_spec, fold_spec],
        out_shape=[
            jax.ShapeDtypeStruct((2, N, F), jnp.float32),
            jax.ShapeDtypeStruct((2, N, 16), jnp.float32),
            jax.ShapeDtypeStruct((2, N, H1), jnp.float32),
        ],
    )(s1, fold1, Wm1_in, bm1_in.reshape(1, H1), Wm1_out,
      bm1_out.reshape(1, H1), Wm1_l, bm1_l.reshape(1, 1),
      W_fc2_in, av2i, W_fc2_out, av2o)

    s2 = _edge_pass(feat2.reshape(2 * N, F), scal2.reshape(2 * N, 16),
                    typ2.reshape(2 * MP, F), zeros_nf, eidx)

    h_prime = pl.pallas_call(
        _tc_final,
        grid=(NB,),
        in_specs=[feat_spec, fold_spec, x_spec,
                  w1spec, b1spec, w1spec, b1spec, wlspec, bl_spec, w1spec],
        out_specs=pl.BlockSpec((BN, H1), lambda i: (i, 0)),
        out_shape=jax.ShapeDtypeStruct((N, H1), jnp.float32),
    )(s2, fold2, x, Wm2_in, bm2_in.reshape(1, H1), Wm2_out,
      bm2_out.reshape(1, H1), Wm2_l, bm2_l.reshape(1, 1), W_ent)

    return h_prime
